# baseline (speedup 1.0000x reference)
import functools

import jax
import jax.numpy as jnp
import numpy as np
from jax import lax
from jax.experimental import pallas as pl
from jax.experimental.pallas import tpu as pltpu
from jax.experimental.pallas import tpu_sc as plsc

B, N, D, C, G = 32, 8192, 256, 91, 4
K = 300
L = 16
NA = 19
KPAD = NA * L
NCHUNK = N // L
_SIGN = np.uint32(0x80000000)

MROWS = N * D // 128
LROWS = N * C // 128
GROWS = N * G // 128
PA = 160
PB = KPAD - PA
COV = 2 * PA


def _keys_body(logits_ref, keys_ref):
    s = jnp.max(logits_ref[...], axis=-1)
    bits = lax.bitcast_convert_type(s, jnp.uint32)
    neg = bits >= _SIGN
    keys_ref[...] = jnp.where(neg, ~bits + np.uint32(1), bits | _SIGN)


_BB, _BN = 8, 1024


def _compute_keys(logits):
    return pl.pallas_call(
        _keys_body,
        grid=(B // _BB, N // _BN),
        in_specs=[pl.BlockSpec((_BB, _BN, C), lambda b, n: (b, n, 0))],
        out_specs=pl.BlockSpec((_BB, _BN), lambda b, n: (b, n)),
        out_shape=jax.ShapeDtypeStruct((B, N), jnp.uint32),
    )(logits)


_mesh = plsc.VectorSubcoreMesh(core_axis_name="c", subcore_axis_name="s",
                               num_cores=2, num_subcores=16)


@functools.partial(
    pl.kernel,
    out_type=(
        jax.ShapeDtypeStruct((B, 2 * K, 128), jnp.float32),
        jax.ShapeDtypeStruct((B, K * C), jnp.float32),
        jax.ShapeDtypeStruct((B, K * G), jnp.float32),
    ),
    mesh=_mesh,
    scratch_types=[
        pltpu.VMEM((N,), jnp.uint32),
        pltpu.VMEM((KPAD + L,), jnp.int32),
        pltpu.VMEM((KPAD + L,), jnp.int32),
        pltpu.VMEM((KPAD + L,), jnp.int32),
        pltpu.VMEM((KPAD,), jnp.int32),
        pltpu.VMEM((COV,), jnp.int32),
        pltpu.VMEM((COV, 128), jnp.float32),
        pltpu.VMEM((KPAD * C,), jnp.float32),
        pltpu.VMEM((KPAD * G,), jnp.float32),
        pltpu.SemaphoreType.DMA,
    ],
    compiler_params=pltpu.CompilerParams(use_tc_tiling_on_sc=False,
                                         needs_layout_passes=False),
)
def _sc_select(keys_hbm, mem_hbm, log_hbm, geo_hbm,
               out_mem, out_log, out_geo,
               keys_v, ak_v, ai_v, ei_v, fi_v, cidx_v, cover_v,
               glog_v, ggeo_v, sem):
    b = lax.axis_index("s") * 2 + lax.axis_index("c")
    iota = lax.iota(jnp.int32, L)
    zero_i = jnp.zeros((L,), jnp.int32)

    pltpu.sync_copy(keys_hbm.at[b], keys_v)

    def _bit_step(i, tv):
        sh = (np.int32(31) - i).astype(jnp.uint32)
        cand = tv | (np.uint32(1) << sh)

        def chunk(ci, cnt):
            for u in range(8):
                kk = keys_v[pl.ds((ci * 8 + u) * L, L)]
                cnt = cnt + plsc.all_reduce_population_count(kk >= cand)
            return cnt

        cnt = lax.fori_loop(0, NCHUNK // 8, chunk, zero_i)
        return jnp.where(cnt >= K, cand, tv)

    Tv = lax.fori_loop(0, 32, _bit_step, jnp.zeros((L,), jnp.uint32))

    def _compact(i, carry):
        cg, ce = carry
        kk = keys_v[pl.ds(i * L, L)]
        idxv = iota + i * L
        m_gt = kk > Tv
        m_eq = kk == Tv
        pos_g = cg + plsc.cumsum(m_gt.astype(jnp.int32)) - 1
        pos_e = ce + plsc.cumsum(m_eq.astype(jnp.int32)) - 1
        plsc.store_scatter(ak_v, [pos_g], plsc.bitcast(kk, jnp.int32),
                           mask=m_gt)
        plsc.store_scatter(ai_v, [pos_g], idxv, mask=m_gt)
        plsc.store_scatter(ei_v, [pos_e], idxv,
                           mask=m_eq & (pos_e < KPAD + L))
        return (cg + plsc.all_reduce_population_count(m_gt),
                ce + plsc.all_reduce_population_count(m_eq))

    cgv, _ = lax.fori_loop(0, NCHUNK, _compact, (zero_i, zero_i))

    for j in range(NA):
        sl = pl.ds(j * L, L)
        p = iota + j * L
        m_pad = p >= cgv
        ak_v[sl] = jnp.where(m_pad, np.int32(0), ak_v[sl])
        ai_v[sl] = jnp.where(m_pad, np.int32(0x40000000) + p, ai_v[sl])

    def _rank_chunk(a, _):
        kt = plsc.bitcast(ak_v[pl.ds(a * L, L)], jnp.uint32)
        it = ai_v[pl.ds(a * L, L)]

        def _src(bs, rank):
            base = bs * L
            for r in range(L):
                pos = base + ((iota + r) & (L - 1))
                ksr = plsc.bitcast(plsc.load_gather(ak_v, [pos]), jnp.uint32)
                isr = plsc.load_gather(ai_v, [pos])
                beat = (ksr > kt) | ((ksr == kt) & (isr < it))
                rank = rank + beat.astype(jnp.int32)
            return rank

        rank = lax.fori_loop(0, NA, _src, zero_i)
        plsc.store_scatter(fi_v, [rank], it, mask=rank < KPAD)
        return 0

    lax.fori_loop(0, NA, _rank_chunk, 0)

    for j in range(NA):
        p = iota + j * L
        m = p >= cgv
        ep = jnp.maximum(p - cgv, 0)
        ev = plsc.load_gather(ei_v, [ep]) & np.int32(N - 1)
        sl = pl.ds(j * L, L)
        fi_v[sl] = jnp.where(m, ev, fi_v[sl])


    def _mem_cidx(q, carry, base, nsel):
        t = q * L + iota
        fj = fi_v[pl.ds(base + q * L, L)]
        msk = t < nsel
        plsc.store_scatter(cidx_v, [2 * t], 2 * fj, mask=msk)
        plsc.store_scatter(cidx_v, [2 * t + 1], 2 * fj + 1, mask=msk)
        return 0

    lax.fori_loop(0, PA // L, functools.partial(_mem_cidx, base=0, nsel=PA),
                  0)
    pltpu.async_copy(mem_hbm.at[b].at[cidx_v], cover_v, sem).wait()
    pltpu.sync_copy(cover_v, out_mem.at[b].at[pl.ds(0, COV)])

    lax.fori_loop(0, 9, functools.partial(_mem_cidx, base=PA, nsel=K - PA), 0)
    pltpu.async_copy(mem_hbm.at[b].at[cidx_v.at[pl.ds(0, 2 * (K - PA))]],
                     cover_v.at[pl.ds(0, 2 * (K - PA))], sem).wait()
    pltpu.sync_copy(cover_v.at[pl.ds(0, 2 * (K - PA))],
                    out_mem.at[b].at[pl.ds(COV, 2 * (K - PA))])

    def _log_cidx(q, carry, base, nsel):
        t = q * L + iota
        fj = fi_v[pl.ds(base + q * L, L)]
        s = fj * C
        r0 = s >> np.int32(7)
        r1 = jnp.minimum(r0 + 1, LROWS - 1)
        msk = t < nsel
        plsc.store_scatter(cidx_v, [2 * t], r0, mask=msk)
        plsc.store_scatter(cidx_v, [2 * t + 1], r1, mask=msk)
        return 0

    def _log_extract(q, carry, base):
        lt = q * L + iota
        t = base + lt
        fj = fi_v[pl.ds(base + q * L, L)]
        src = lt * 256 + (fj * C & np.int32(127))
        dst = t * C
        for c in range(C):
            so = src + c
            v = plsc.load_gather(
                cover_v, [so >> np.int32(7),
                          so & np.int32(127)])
            plsc.store_scatter(glog_v, [dst + c], v)
        return 0

    lax.fori_loop(0, PA // L, functools.partial(_log_cidx, base=0, nsel=PA),
                  0)
    pltpu.async_copy(log_hbm.at[b].at[cidx_v], cover_v, sem).wait()
    lax.fori_loop(0, PA // L, functools.partial(_log_extract, base=0), 0)

    lax.fori_loop(0, PB // L, functools.partial(_log_cidx, base=PA, nsel=PB),
                  0)
    pltpu.async_copy(log_hbm.at[b].at[cidx_v.at[pl.ds(0, 2 * PB)]],
                     cover_v.at[pl.ds(0, 2 * PB)], sem).wait()
    lax.fori_loop(0, PB // L, functools.partial(_log_extract, base=PA), 0)

    pltpu.sync_copy(glog_v.at[pl.ds(0, K * C)], out_log.at[b])

    def _geo_cidx(q, _):
        t = q * L + iota
        fj = fi_v[pl.ds(q * L, L)]
        plsc.store_scatter(cidx_v, [t], fj >> np.int32(5))
        return 0

    lax.fori_loop(0, NA, _geo_cidx, 0)
    zpad = jnp.zeros((L,), jnp.int32)
    cidx_v[pl.ds(KPAD, L)] = zpad
    pltpu.async_copy(geo_hbm.at[b].at[cidx_v], cover_v, sem).wait()

    def _geo_extract(q, _):
        t = q * L + iota
        fj = fi_v[pl.ds(q * L, L)]
        src = t * 128 + (fj & np.int32(31)) * G
        dst = t * G
        for c in range(G):
            so = src + c
            v = plsc.load_gather(
                cover_v, [so >> np.int32(7),
                          so & np.int32(127)])
            plsc.store_scatter(ggeo_v, [dst + c], v)
        return 0

    lax.fori_loop(0, NA, _geo_extract, 0)
    pltpu.sync_copy(ggeo_v.at[pl.ds(0, K * G)], out_geo.at[b])


def kernel(memory, query_class_logits, query_geometries_unactivated):
    keys = _compute_keys(query_class_logits)
    memv = memory.reshape(B, MROWS, 128)
    logv = query_class_logits.reshape(B, LROWS, 128)
    geov = query_geometries_unactivated.reshape(B, GROWS, 128)
    tm, tl, tg = _sc_select(keys, memv, logv, geov)
    return (tm.reshape(B, K, D),
            tl.reshape(B, K, C),
            tg.reshape(B, K, G))


# speedup vs baseline: 1.7934x; 1.7934x over previous
import functools

import jax
import jax.numpy as jnp
import numpy as np
from jax import lax
from jax.experimental import pallas as pl
from jax.experimental.pallas import tpu as pltpu
from jax.experimental.pallas import tpu_sc as plsc

B, N, D, C, G = 32, 8192, 256, 91, 4
K = 300
L = 16
NA = 19
KPAD = NA * L
NCHUNK = N // L
HALF = 152
_SIGN = np.uint32(0x80000000)


def _keys_body(logits_ref, keys_ref, padlog_ref):
    blk = logits_ref[...]
    s = jnp.max(blk, axis=-1)
    bits = lax.bitcast_convert_type(s, jnp.uint32)
    neg = bits >= _SIGN
    keys_ref[...] = jnp.where(neg, ~bits + np.uint32(1), bits | _SIGN)
    padlog_ref[:, :, :C] = blk


_BB, _BN = 8, 512


def _compute_keys(logits):
    return pl.pallas_call(
        _keys_body,
        grid=(B // _BB, N // _BN),
        in_specs=[pl.BlockSpec((_BB, _BN, C), lambda b, n: (b, n, 0))],
        out_specs=[pl.BlockSpec((_BB, _BN), lambda b, n: (b, n)),
                   pl.BlockSpec((_BB, _BN, 128), lambda b, n: (b, n, 0))],
        out_shape=[jax.ShapeDtypeStruct((B, N), jnp.uint32),
                   jax.ShapeDtypeStruct((B, N, 128), jnp.float32)],
    )(logits)


_mesh = plsc.VectorSubcoreMesh(core_axis_name="c", subcore_axis_name="s",
                               num_cores=2, num_subcores=16)


@functools.partial(
    pl.kernel,
    out_type=(
        jax.ShapeDtypeStruct((B, KPAD, D), jnp.float32),
        jax.ShapeDtypeStruct((B, 27392), jnp.float32),
        jax.ShapeDtypeStruct((B, 1280), jnp.float32),
    ),
    mesh=_mesh,
    scratch_types=[
        pltpu.VMEM((N,), jnp.uint32),
        pltpu.VMEM((KPAD + L,), jnp.int32),
        pltpu.VMEM((KPAD + L,), jnp.int32),
        pltpu.VMEM((KPAD + L,), jnp.int32),
        pltpu.VMEM((KPAD,), jnp.int32),
        pltpu.VMEM((HALF, D), jnp.float32),
        pltpu.VMEM((HALF, 128), jnp.float32),
        pltpu.VMEM((KPAD * C,), jnp.float32),
        pltpu.VMEM((N,), jnp.float32),
        pltpu.VMEM((1280,), jnp.float32),
        pltpu.SemaphoreType.DMA,
    ],
    compiler_params=pltpu.CompilerParams(needs_layout_passes=False),
)
def _sc_select(keys_hbm, mem_hbm, padlog_hbm, geo0_hbm, geo1_hbm, geo2_hbm,
               geo3_hbm,
               out_mem, out_log, out_geo,
               keys_v, ak_v, ai_v, ei_v, fi_v, gmem_v, cover_v,
               glog_v, gstage_v, ggeo_v, sem):
    b = lax.axis_index("s") * 2 + lax.axis_index("c")
    iota = lax.iota(jnp.int32, L)
    zero_i = jnp.zeros((L,), jnp.int32)

    pltpu.sync_copy(keys_hbm.at[b], keys_v)

    def _bit_step(i, tv):
        sh = (np.int32(31) - i).astype(jnp.uint32)
        cand = tv | (np.uint32(1) << sh)

        def chunk(ci, cnt):
            for u in range(8):
                kk = keys_v[pl.ds((ci * 8 + u) * L, L)]
                cnt = cnt + plsc.all_reduce_population_count(kk >= cand)
            return cnt

        cnt = lax.fori_loop(0, NCHUNK // 8, chunk, zero_i)
        return jnp.where(cnt >= K, cand, tv)

    Tv = lax.fori_loop(0, 32, _bit_step, jnp.zeros((L,), jnp.uint32))

    def _compact(i, carry):
        cg, ce = carry
        kk = keys_v[pl.ds(i * L, L)]
        idxv = iota + i * L
        m_gt = kk > Tv
        m_eq = kk == Tv
        pos_g = cg + plsc.cumsum(m_gt.astype(jnp.int32)) - 1
        pos_e = ce + plsc.cumsum(m_eq.astype(jnp.int32)) - 1
        plsc.store_scatter(ak_v, [pos_g], plsc.bitcast(kk, jnp.int32),
                           mask=m_gt)
        plsc.store_scatter(ai_v, [pos_g], idxv, mask=m_gt)
        plsc.store_scatter(ei_v, [pos_e], idxv,
                           mask=m_eq & (pos_e < KPAD + L))
        return (cg + plsc.all_reduce_population_count(m_gt),
                ce + plsc.all_reduce_population_count(m_eq))

    cgv, _ = lax.fori_loop(0, NCHUNK, _compact, (zero_i, zero_i))

    for j in range(NA):
        sl = pl.ds(j * L, L)
        p = iota + j * L
        m_pad = p >= cgv
        ak_v[sl] = jnp.where(m_pad, np.int32(0), ak_v[sl])
        ai_v[sl] = jnp.where(m_pad, np.int32(0x40000000) + p, ai_v[sl])

    def _rank_chunk(a, _):
        kt = plsc.bitcast(ak_v[pl.ds(a * L, L)], jnp.uint32)
        it = ai_v[pl.ds(a * L, L)]

        def _src(bs, rank):
            base = bs * L
            for r in range(L):
                pos = base + ((iota + r) & (L - 1))
                ksr = plsc.bitcast(plsc.load_gather(ak_v, [pos]), jnp.uint32)
                isr = plsc.load_gather(ai_v, [pos])
                beat = (ksr > kt) | ((ksr == kt) & (isr < it))
                rank = rank + beat.astype(jnp.int32)
            return rank

        rank = lax.fori_loop(0, NA, _src, zero_i)
        plsc.store_scatter(fi_v, [rank], it, mask=rank < KPAD)
        return 0

    lax.fori_loop(0, NA, _rank_chunk, 0)

    for j in range(NA):
        p = iota + j * L
        m = p >= cgv
        ep = jnp.maximum(p - cgv, 0)
        ev = plsc.load_gather(ei_v, [ep]) & np.int32(N - 1)
        sl = pl.ds(j * L, L)
        fi_v[sl] = jnp.where(m, ev, fi_v[sl])


    for h in range(2):
        idx_ref = fi_v.at[pl.ds(h * HALF, HALF)]
        pltpu.async_copy(mem_hbm.at[b].at[idx_ref], gmem_v, sem).wait()
        pltpu.sync_copy(gmem_v, out_mem.at[b].at[pl.ds(h * HALF, HALF)])

    for h in range(2):
        idx_ref = fi_v.at[pl.ds(h * HALF, HALF)]
        pltpu.async_copy(padlog_hbm.at[b].at[idx_ref], cover_v, sem).wait()

        def _log_extract(q, _, base=h * HALF):
            lt = q * L + iota
            t = base + lt
            src = lt * 128
            dst = t * C
            msk = t < K
            for c in range(C):
                so = jnp.minimum(src + c, np.int32(HALF * 128 - 1))
                v = plsc.load_gather(
                    cover_v, [so >> np.int32(7), so & np.int32(127)])
                plsc.store_scatter(glog_v, [(dst + c) & np.int32(0x7FFF)],
                                   v, mask=msk)
            return 0

        lax.fori_loop(0, HALF // L + 1, _log_extract, 0)

    pltpu.sync_copy(glog_v.at[pl.ds(0, 27392)], out_log.at[b])

    for g, plane in enumerate((geo0_hbm, geo1_hbm, geo2_hbm, geo3_hbm)):
        pltpu.sync_copy(plane.at[b], gstage_v)

        def _geo_extract(q, _, g=g):
            t = q * L + iota
            fj = fi_v[pl.ds(q * L, L)]
            v = plsc.load_gather(gstage_v, [fj])
            plsc.store_scatter(ggeo_v, [t * G + g], v, mask=t < K)
            return 0

        lax.fori_loop(0, NA, _geo_extract, 0)

    pltpu.sync_copy(ggeo_v, out_geo.at[b])


def kernel(memory, query_class_logits, query_geometries_unactivated):
    keys, padlog = _compute_keys(query_class_logits)
    geo = query_geometries_unactivated
    tm, tl, tg = _sc_select(keys, memory, padlog,
                            geo[:, :, 0], geo[:, :, 1],
                            geo[:, :, 2], geo[:, :, 3])
    return (tm[:, :K],
            tl[:, :K * C].reshape(B, K, C),
            tg[:, :K * G].reshape(B, K, G))


# speedup vs baseline: 1.7972x; 1.0021x over previous
import functools

import jax
import jax.numpy as jnp
import numpy as np
from jax import lax
from jax.experimental import pallas as pl
from jax.experimental.pallas import tpu as pltpu
from jax.experimental.pallas import tpu_sc as plsc

B, N, D, C, G = 32, 8192, 256, 91, 4
K = 300
L = 16
NA = 19
KPAD = NA * L
NCHUNK = N // L
HALF = 152
_SIGN = np.uint32(0x80000000)


def _keys_body(logits_ref, keys_ref, padlog_ref):
    blk = logits_ref[...]
    s = jnp.max(blk, axis=-1)
    bits = lax.bitcast_convert_type(s, jnp.uint32)
    neg = bits >= _SIGN
    keys_ref[...] = jnp.where(neg, ~bits + np.uint32(1), bits | _SIGN)
    padlog_ref[...] = lax.pad(blk, np.float32(0),
                              ((0, 0, 0), (0, 0, 0), (0, 128 - C, 0)))


_BB, _BN = 8, 512


def _compute_keys(logits):
    return pl.pallas_call(
        _keys_body,
        grid=(B // _BB, N // _BN),
        in_specs=[pl.BlockSpec((_BB, _BN, C), lambda b, n: (b, n, 0))],
        out_specs=[pl.BlockSpec((_BB, _BN), lambda b, n: (b, n)),
                   pl.BlockSpec((_BB, _BN, 128), lambda b, n: (b, n, 0))],
        out_shape=[jax.ShapeDtypeStruct((B, N), jnp.uint32),
                   jax.ShapeDtypeStruct((B, N, 128), jnp.float32)],
    )(logits)


_mesh = plsc.VectorSubcoreMesh(core_axis_name="c", subcore_axis_name="s",
                               num_cores=2, num_subcores=16)


@functools.partial(
    pl.kernel,
    out_type=(
        jax.ShapeDtypeStruct((B, KPAD, D), jnp.float32),
        jax.ShapeDtypeStruct((B, 27392), jnp.float32),
        jax.ShapeDtypeStruct((B, 1280), jnp.float32),
    ),
    mesh=_mesh,
    scratch_types=[
        pltpu.VMEM((N,), jnp.uint32),
        pltpu.VMEM((KPAD + L,), jnp.int32),
        pltpu.VMEM((KPAD + L,), jnp.int32),
        pltpu.VMEM((KPAD + L,), jnp.int32),
        pltpu.VMEM((KPAD,), jnp.int32),
        pltpu.VMEM((HALF, D), jnp.float32),
        pltpu.VMEM((HALF, 128), jnp.float32),
        pltpu.VMEM((KPAD * C,), jnp.float32),
        pltpu.VMEM((N,), jnp.float32),
        pltpu.VMEM((1280,), jnp.float32),
        pltpu.SemaphoreType.DMA,
    ],
    compiler_params=pltpu.CompilerParams(needs_layout_passes=False),
)
def _sc_select(keys_hbm, mem_hbm, padlog_hbm, geo0_hbm, geo1_hbm, geo2_hbm,
               geo3_hbm,
               out_mem, out_log, out_geo,
               keys_v, ak_v, ai_v, ei_v, fi_v, gmem_v, cover_v,
               glog_v, gstage_v, ggeo_v, sem):
    b = lax.axis_index("s") * 2 + lax.axis_index("c")
    iota = lax.iota(jnp.int32, L)
    zero_i = jnp.zeros((L,), jnp.int32)

    pltpu.sync_copy(keys_hbm.at[b], keys_v)

    def _bit_step(i, tv):
        sh = (np.int32(31) - i).astype(jnp.uint32)
        cand = tv | (np.uint32(1) << sh)

        def chunk(ci, cnt):
            for u in range(8):
                kk = keys_v[pl.ds((ci * 8 + u) * L, L)]
                cnt = cnt + plsc.all_reduce_population_count(kk >= cand)
            return cnt

        cnt = lax.fori_loop(0, NCHUNK // 8, chunk, zero_i)
        return jnp.where(cnt >= K, cand, tv)

    Tv = lax.fori_loop(0, 32, _bit_step, jnp.zeros((L,), jnp.uint32))

    def _compact(i, carry):
        cg, ce = carry
        kk = keys_v[pl.ds(i * L, L)]
        idxv = iota + i * L
        m_gt = kk > Tv
        m_eq = kk == Tv
        pos_g = cg + plsc.cumsum(m_gt.astype(jnp.int32)) - 1
        pos_e = ce + plsc.cumsum(m_eq.astype(jnp.int32)) - 1
        plsc.store_scatter(ak_v, [pos_g], plsc.bitcast(kk, jnp.int32),
                           mask=m_gt)
        plsc.store_scatter(ai_v, [pos_g], idxv, mask=m_gt)
        plsc.store_scatter(ei_v, [pos_e], idxv,
                           mask=m_eq & (pos_e < KPAD + L))
        return (cg + plsc.all_reduce_population_count(m_gt),
                ce + plsc.all_reduce_population_count(m_eq))

    cgv, _ = lax.fori_loop(0, NCHUNK, _compact, (zero_i, zero_i))

    for j in range(NA):
        sl = pl.ds(j * L, L)
        p = iota + j * L
        m_pad = p >= cgv
        ak_v[sl] = jnp.where(m_pad, np.int32(0), ak_v[sl])
        ai_v[sl] = jnp.where(m_pad, np.int32(0x40000000) + p, ai_v[sl])

    def _rank_chunk(a, _):
        kt = plsc.bitcast(ak_v[pl.ds(a * L, L)], jnp.uint32)
        it = ai_v[pl.ds(a * L, L)]

        def _src(bs, rank):
            base = bs * L
            for r in range(L):
                pos = base + ((iota + r) & (L - 1))
                ksr = plsc.bitcast(plsc.load_gather(ak_v, [pos]), jnp.uint32)
                isr = plsc.load_gather(ai_v, [pos])
                beat = (ksr > kt) | ((ksr == kt) & (isr < it))
                rank = rank + beat.astype(jnp.int32)
            return rank

        rank = lax.fori_loop(0, NA, _src, zero_i)
        plsc.store_scatter(fi_v, [rank], it, mask=rank < KPAD)
        return 0

    lax.fori_loop(0, NA, _rank_chunk, 0)

    for j in range(NA):
        p = iota + j * L
        m = p >= cgv
        ep = jnp.maximum(p - cgv, 0)
        ev = plsc.load_gather(ei_v, [ep]) & np.int32(N - 1)
        sl = pl.ds(j * L, L)
        fi_v[sl] = jnp.where(m, ev, fi_v[sl])


    for h in range(2):
        idx_ref = fi_v.at[pl.ds(h * HALF, HALF)]
        pltpu.async_copy(mem_hbm.at[b].at[idx_ref], gmem_v, sem).wait()
        pltpu.sync_copy(gmem_v, out_mem.at[b].at[pl.ds(h * HALF, HALF)])

    for h in range(2):
        idx_ref = fi_v.at[pl.ds(h * HALF, HALF)]
        pltpu.async_copy(padlog_hbm.at[b].at[idx_ref], cover_v, sem).wait()

        def _log_extract(q, _, base=h * HALF):
            lt = q * L + iota
            t = base + lt
            src = lt * 128
            dst = t * C
            msk = t < K
            for c in range(C):
                so = jnp.minimum(src + c, np.int32(HALF * 128 - 1))
                v = plsc.load_gather(
                    cover_v, [so >> np.int32(7), so & np.int32(127)])
                plsc.store_scatter(glog_v, [(dst + c) & np.int32(0x7FFF)],
                                   v, mask=msk)
            return 0

        lax.fori_loop(0, HALF // L + 1, _log_extract, 0)

    pltpu.sync_copy(glog_v.at[pl.ds(0, 27392)], out_log.at[b])

    for g, plane in enumerate((geo0_hbm, geo1_hbm, geo2_hbm, geo3_hbm)):
        pltpu.sync_copy(plane.at[b], gstage_v)

        def _geo_extract(q, _, g=g):
            t = q * L + iota
            fj = fi_v[pl.ds(q * L, L)]
            v = plsc.load_gather(gstage_v, [fj])
            plsc.store_scatter(ggeo_v, [t * G + g], v, mask=t < K)
            return 0

        lax.fori_loop(0, NA, _geo_extract, 0)

    pltpu.sync_copy(ggeo_v, out_geo.at[b])


def kernel(memory, query_class_logits, query_geometries_unactivated):
    keys, padlog = _compute_keys(query_class_logits)
    geo = query_geometries_unactivated
    tm, tl, tg = _sc_select(keys, memory, padlog,
                            geo[:, :, 0], geo[:, :, 1],
                            geo[:, :, 2], geo[:, :, 3])
    return (tm[:, :K],
            tl[:, :K * C].reshape(B, K, C),
            tg[:, :K * G].reshape(B, K, G))
